# initial kernel scaffold (unmeasured)
import jax
import jax.numpy as jnp
from jax import lax
from jax.experimental import pallas as pl
from jax.experimental.pallas import tpu as pltpu

N_DEV = 4


def kernel(x, Win0, Wout0, Win1, Wout1, Win2, Wout2):
    m, d = x.shape
    M = N_DEV * m

    win0, win1, win2 = (w.astype(jnp.bfloat16) for w in (Win0, Win1, Win2))
    wout0, wout1, wout2 = (w.astype(jnp.bfloat16) for w in (Wout0, Wout1, Wout2))

    def body(x_ref, w_in0, w_out0, w_in1, w_out1, w_in2, w_out2, out_ref,
             xg, sbuf, rbuf, ag_send, ag_recv, ar_send, ar_recv):
        my = lax.axis_index("i")

        bar = pltpu.get_barrier_semaphore()
        for k in range(1, N_DEV):
            pl.semaphore_signal(
                bar, inc=1,
                device_id=((my + k) % N_DEV,),
                device_id_type=pl.DeviceIdType.MESH,
            )
        pl.semaphore_wait(bar, N_DEV - 1)

        xg[pl.ds(my * m, m), :] = x_ref[:, :].astype(jnp.bfloat16)
        ag_rdmas = []
        for k in range(1, N_DEV):
            dst = (my + k) % N_DEV
            r = pltpu.make_async_remote_copy(
                src_ref=xg.at[pl.ds(my * m, m), :],
                dst_ref=xg.at[pl.ds(my * m, m), :],
                send_sem=ag_send.at[k - 1],
                recv_sem=ag_recv.at[k - 1],
                device_id=(dst,),
                device_id_type=pl.DeviceIdType.MESH,
            )
            r.start()
            ag_rdmas.append(r)
        for r in ag_rdmas:
            r.wait()

        xfull = xg[:, :]
        total = None
        for l, (w_in, w_out) in enumerate(
            ((w_in0, w_out0), (w_in1, w_out1), (w_in2, w_out2))
        ):
            hidden = jnp.maximum(
                jnp.dot(xfull, w_in[:, :], preferred_element_type=jnp.float32),
                0.0,
            )
            partial = jnp.dot(
                hidden.astype(jnp.bfloat16), w_out[:, :],
                preferred_element_type=jnp.float32,
            )
            sbuf[:, :] = partial.astype(jnp.bfloat16)
            rdmas = []
            for k in range(1, N_DEV):
                dst = (my + k) % N_DEV
                r = pltpu.make_async_remote_copy(
                    src_ref=sbuf,
                    dst_ref=rbuf.at[l, k - 1],
                    send_sem=ar_send.at[l, k - 1],
                    recv_sem=ar_recv.at[l, k - 1],
                    device_id=(dst,),
                    device_id_type=pl.DeviceIdType.MESH,
                )
                r.start()
                rdmas.append(r)
            for r in rdmas:
                r.wait()
            total = partial
            for k in range(1, N_DEV):
                total = total + rbuf[l, k - 1].astype(jnp.float32)
            xfull = total.astype(jnp.bfloat16)

        out_ref[:, :] = lax.dynamic_slice(total, (my * m, 0), (m, d))

    return pl.pallas_call(
        body,
        out_shape=jax.ShapeDtypeStruct((m, d), jnp.float32),
        in_specs=[pl.BlockSpec(memory_space=pltpu.VMEM)] * 7,
        out_specs=pl.BlockSpec(memory_space=pltpu.VMEM),
        scratch_shapes=[
            pltpu.VMEM((M, d), jnp.bfloat16),
            pltpu.VMEM((M, d), jnp.bfloat16),
            pltpu.VMEM((3, N_DEV - 1, M, d), jnp.bfloat16),
            pltpu.SemaphoreType.DMA((N_DEV - 1,)),
            pltpu.SemaphoreType.DMA((N_DEV - 1,)),
            pltpu.SemaphoreType.DMA((3, N_DEV - 1)),
            pltpu.SemaphoreType.DMA((3, N_DEV - 1)),
        ],
        compiler_params=pltpu.CompilerParams(collective_id=0),
    )(x, win0, wout0, win1, wout1, win2, wout2)


# baseline (device time: 79161 ns/iter reference)
import jax
import jax.numpy as jnp
from jax import lax
from jax.experimental import pallas as pl
from jax.experimental.pallas import tpu as pltpu

N_DEV = 4


def kernel(x, Win0, Wout0, Win1, Wout1, Win2, Wout2):
    m, d = x.shape
    M = N_DEV * m

    win0, win1, win2 = (w.astype(jnp.bfloat16) for w in (Win0, Win1, Win2))
    wout0, wout1, wout2 = (w.astype(jnp.bfloat16) for w in (Wout0, Wout1, Wout2))

    def body(x_ref, w_in0, w_out0, w_in1, w_out1, w_in2, w_out2, out_ref,
             xg, sbuf, rbuf, acc, ag_send, ag_recv, ar_send, ar_recv):
        my = lax.axis_index("i")

        bar = pltpu.get_barrier_semaphore()
        for k in range(1, N_DEV):
            pl.semaphore_signal(
                bar, inc=1,
                device_id=((my + k) % N_DEV,),
                device_id_type=pl.DeviceIdType.MESH,
            )
        pl.semaphore_wait(bar, N_DEV - 1)

        xg[pl.ds(my * m, m), :] = x_ref[:, :].astype(jnp.bfloat16)
        ag_rdmas = []
        for k in range(1, N_DEV):
            dst = (my + k) % N_DEV
            r = pltpu.make_async_remote_copy(
                src_ref=xg.at[pl.ds(my * m, m), :],
                dst_ref=xg.at[pl.ds(my * m, m), :],
                send_sem=ag_send.at[k - 1],
                recv_sem=ag_recv.at[k - 1],
                device_id=(dst,),
                device_id_type=pl.DeviceIdType.MESH,
            )
            r.start()
            ag_rdmas.append(r)
        for r in ag_rdmas:
            r.wait()

        xfull = xg[:, :]
        total = None
        for l, (w_in, w_out) in enumerate(
            ((w_in0, w_out0), (w_in1, w_out1), (w_in2, w_out2))
        ):
            hidden = jnp.maximum(
                jnp.dot(xfull, w_in[:, :], preferred_element_type=jnp.float32),
                0.0,
            )
            partial = jnp.dot(
                hidden.astype(jnp.bfloat16), w_out[:, :],
                preferred_element_type=jnp.float32,
            )
            sbuf[:, :] = partial.astype(jnp.bfloat16)
            rdmas = []
            for k in range(1, N_DEV):
                dst = (my + k) % N_DEV
                r = pltpu.make_async_remote_copy(
                    src_ref=sbuf,
                    dst_ref=rbuf.at[l, k - 1],
                    send_sem=ar_send.at[l, k - 1],
                    recv_sem=ar_recv.at[l, k - 1],
                    device_id=(dst,),
                    device_id_type=pl.DeviceIdType.MESH,
                )
                r.start()
                rdmas.append(r)
            for r in rdmas:
                r.wait()
            total = partial
            for k in range(1, N_DEV):
                total = total + rbuf[l, k - 1].astype(jnp.float32)
            xfull = total.astype(jnp.bfloat16)

        acc[:, :] = total
        out_ref[:, :] = acc[pl.ds(my * m, m), :]

    return pl.pallas_call(
        body,
        out_shape=jax.ShapeDtypeStruct((m, d), jnp.float32),
        in_specs=[pl.BlockSpec(memory_space=pltpu.VMEM)] * 7,
        out_specs=pl.BlockSpec(memory_space=pltpu.VMEM),
        scratch_shapes=[
            pltpu.VMEM((M, d), jnp.bfloat16),
            pltpu.VMEM((M, d), jnp.bfloat16),
            pltpu.VMEM((3, N_DEV - 1, M, d), jnp.bfloat16),
            pltpu.VMEM((M, d), jnp.float32),
            pltpu.SemaphoreType.DMA((N_DEV - 1,)),
            pltpu.SemaphoreType.DMA((N_DEV - 1,)),
            pltpu.SemaphoreType.DMA((3, N_DEV - 1)),
            pltpu.SemaphoreType.DMA((3, N_DEV - 1)),
        ],
        compiler_params=pltpu.CompilerParams(collective_id=0),
    )(x, win0, wout0, win1, wout1, win2, wout2)


# device time: 45782 ns/iter; 1.7291x vs baseline; 1.7291x over previous
import jax
import jax.numpy as jnp
from jax import lax
from jax.experimental import pallas as pl
from jax.experimental.pallas import tpu as pltpu

N_DEV = 4
N_LAYERS = 3


def kernel(x, Win0, Wout0, Win1, Wout1, Win2, Wout2):
    m, d = x.shape
    h = Win0.shape[1]
    M = N_DEV * m

    def body(x_ref, w_in0, w_out0, w_in1, w_out1, w_in2, w_out2, out_ref,
             xbuf, psend, rsbuf, winbuf, woutbuf,
             wsem, ag_send, ag_recv, rs_send, rs_recv):
        my = lax.axis_index("i")
        w_in_hbm = (w_in0, w_in1, w_in2)
        w_out_hbm = (w_out0, w_out1, w_out2)

        bar = pltpu.get_barrier_semaphore()
        for k in range(1, N_DEV):
            pl.semaphore_signal(
                bar, inc=1,
                device_id=((my + k) % N_DEV,),
                device_id_type=pl.DeviceIdType.MESH,
            )
        pl.semaphore_wait(bar, N_DEV - 1)

        def start_wload(l):
            pltpu.make_async_copy(w_in_hbm[l], winbuf.at[l % 2], wsem.at[l, 0]).start()
            pltpu.make_async_copy(w_out_hbm[l], woutbuf.at[l % 2], wsem.at[l, 1]).start()

        start_wload(0)
        start_wload(1)

        send_handles = []

        def ag_push(l, src_block):
            for k in range(1, N_DEV):
                r = pltpu.make_async_remote_copy(
                    src_ref=src_block,
                    dst_ref=src_block,
                    send_sem=ag_send.at[l, k - 1],
                    recv_sem=ag_recv.at[l, k - 1],
                    device_id=((my + k) % N_DEV,),
                    device_id_type=pl.DeviceIdType.MESH,
                )
                r.start()
                send_handles.append(r)

        my_block = pl.ds(my * m, m)
        xbuf[0, my_block, :] = x_ref[:, :].astype(jnp.bfloat16)
        ag_push(0, xbuf.at[0, my_block, :])

        xblk0 = x_ref[:, :].astype(jnp.bfloat16)
        for l in range(N_LAYERS):
            slot = l % 2
            pltpu.make_async_copy(w_in_hbm[l], winbuf.at[slot], wsem.at[l, 0]).wait()
            pltpu.make_async_copy(w_out_hbm[l], woutbuf.at[slot], wsem.at[l, 1]).wait()
            w_in = winbuf[slot, :, :].astype(jnp.bfloat16)
            w_out = woutbuf[slot, :, :].astype(jnp.bfloat16)

            own_partial = None
            for t in range(N_DEV):
                b = (my - t) % N_DEV
                if t == 0:
                    xblk = xblk0
                else:
                    blk = pl.ds(b * m, m)
                    pltpu.make_async_remote_copy(
                        src_ref=xbuf.at[l, blk, :],
                        dst_ref=xbuf.at[l, blk, :],
                        send_sem=ag_send.at[l, t - 1],
                        recv_sem=ag_recv.at[l, t - 1],
                        device_id=(b,),
                        device_id_type=pl.DeviceIdType.MESH,
                    ).wait_recv()
                    xblk = xbuf[l, blk, :]
                hidden = jnp.maximum(
                    jnp.dot(xblk, w_in, preferred_element_type=jnp.float32), 0.0
                )
                partial = jnp.dot(
                    hidden.astype(jnp.bfloat16), w_out,
                    preferred_element_type=jnp.float32,
                )
                if t == 0:
                    own_partial = partial
                else:
                    psend[l, t - 1, :, :] = partial.astype(jnp.bfloat16)
                    r = pltpu.make_async_remote_copy(
                        src_ref=psend.at[l, t - 1],
                        dst_ref=rsbuf.at[l, t - 1],
                        send_sem=rs_send.at[l, t - 1],
                        recv_sem=rs_recv.at[l, t - 1],
                        device_id=(b,),
                        device_id_type=pl.DeviceIdType.MESH,
                    )
                    r.start()
                    send_handles.append(r)

            if l == 0:
                start_wload(2)

            for s in range(N_DEV - 1):
                pltpu.make_async_remote_copy(
                    src_ref=psend.at[l, s],
                    dst_ref=rsbuf.at[l, s],
                    send_sem=rs_send.at[l, s],
                    recv_sem=rs_recv.at[l, s],
                    device_id=(my,),
                    device_id_type=pl.DeviceIdType.MESH,
                ).wait_recv()
                own_partial = own_partial + rsbuf[l, s, :, :].astype(jnp.float32)

            if l < N_LAYERS - 1:
                xblk0 = own_partial.astype(jnp.bfloat16)
                xbuf[l + 1, my_block, :] = xblk0
                ag_push(l + 1, xbuf.at[l + 1, my_block, :])
            else:
                out_ref[:, :] = own_partial

        for r in send_handles:
            r.wait_send()

    return pl.pallas_call(
        body,
        out_shape=jax.ShapeDtypeStruct((m, d), jnp.float32),
        in_specs=[pl.BlockSpec(memory_space=pltpu.VMEM)]
        + [pl.BlockSpec(memory_space=pl.ANY)] * 6,
        out_specs=pl.BlockSpec(memory_space=pltpu.VMEM),
        scratch_shapes=[
            pltpu.VMEM((N_LAYERS, M, d), jnp.bfloat16),
            pltpu.VMEM((N_LAYERS, N_DEV - 1, m, d), jnp.bfloat16),
            pltpu.VMEM((N_LAYERS, N_DEV - 1, m, d), jnp.bfloat16),
            pltpu.VMEM((2, d, h), jnp.float32),
            pltpu.VMEM((2, h, d), jnp.float32),
            pltpu.SemaphoreType.DMA((N_LAYERS, 2)),
            pltpu.SemaphoreType.DMA((N_LAYERS, N_DEV - 1)),
            pltpu.SemaphoreType.DMA((N_LAYERS, N_DEV - 1)),
            pltpu.SemaphoreType.DMA((N_LAYERS, N_DEV - 1)),
            pltpu.SemaphoreType.DMA((N_LAYERS, N_DEV - 1)),
        ],
        compiler_params=pltpu.CompilerParams(
            collective_id=0,
            vmem_limit_bytes=60 * 1024 * 1024,
        ),
    )(x, Win0, Wout0, Win1, Wout1, Win2, Wout2)


# device time: 44493 ns/iter; 1.7792x vs baseline; 1.0290x over previous
import jax
import jax.numpy as jnp
from jax import lax
from jax.experimental import pallas as pl
from jax.experimental.pallas import tpu as pltpu

N_DEV = 4
N_LAYERS = 3


def kernel(x, Win0, Wout0, Win1, Wout1, Win2, Wout2):
    m, d = x.shape
    h = Win0.shape[1]
    M = N_DEV * m

    def body(x_ref, w_in0, w_out0, w_in1, w_out1, w_in2, w_out2, out_ref,
             xbuf, psend, rsbuf, winbuf, woutbuf,
             wsem, ag_send, ag_recv, rs_send, rs_recv):
        my = lax.axis_index("i")
        w_in_hbm = (w_in0, w_in1, w_in2)
        w_out_hbm = (w_out0, w_out1, w_out2)

        def start_wload(l):
            pltpu.make_async_copy(w_in_hbm[l], winbuf.at[l % 2], wsem.at[l, 0]).start()
            pltpu.make_async_copy(w_out_hbm[l], woutbuf.at[l % 2], wsem.at[l, 1]).start()

        start_wload(0)
        start_wload(1)
        my_block = pl.ds(my * m, m)
        xbuf[0, my_block, :] = x_ref[:, :].astype(jnp.bfloat16)

        bar = pltpu.get_barrier_semaphore()
        for k in range(1, N_DEV):
            pl.semaphore_signal(
                bar, inc=1,
                device_id=((my + k) % N_DEV,),
                device_id_type=pl.DeviceIdType.MESH,
            )
        pl.semaphore_wait(bar, N_DEV - 1)

        send_handles = []

        def ag_push(l, src_block):
            for k in range(1, N_DEV):
                r = pltpu.make_async_remote_copy(
                    src_ref=src_block,
                    dst_ref=src_block,
                    send_sem=ag_send.at[l, k - 1],
                    recv_sem=ag_recv.at[l, k - 1],
                    device_id=((my + k) % N_DEV,),
                    device_id_type=pl.DeviceIdType.MESH,
                )
                r.start()
                send_handles.append(r)

        ag_push(0, xbuf.at[0, my_block, :])

        xblk0 = x_ref[:, :].astype(jnp.bfloat16)
        for l in range(N_LAYERS):
            slot = l % 2
            pltpu.make_async_copy(w_in_hbm[l], winbuf.at[slot], wsem.at[l, 0]).wait()
            w_in = winbuf[slot, :, :].astype(jnp.bfloat16)
            pltpu.make_async_copy(w_out_hbm[l], woutbuf.at[slot], wsem.at[l, 1]).wait()
            w_out = woutbuf[slot, :, :].astype(jnp.bfloat16)

            own_partial = None
            for t in range(N_DEV):
                b = (my - t) % N_DEV
                if t == 0:
                    xblk = xblk0
                else:
                    blk = pl.ds(b * m, m)
                    pltpu.make_async_remote_copy(
                        src_ref=xbuf.at[l, blk, :],
                        dst_ref=xbuf.at[l, blk, :],
                        send_sem=ag_send.at[l, t - 1],
                        recv_sem=ag_recv.at[l, t - 1],
                        device_id=(b,),
                        device_id_type=pl.DeviceIdType.MESH,
                    ).wait_recv()
                    xblk = xbuf[l, blk, :]
                hidden = jnp.maximum(
                    jnp.dot(xblk, w_in, preferred_element_type=jnp.float32), 0.0
                )
                partial = jnp.dot(
                    hidden.astype(jnp.bfloat16), w_out,
                    preferred_element_type=jnp.float32,
                )
                if t == 0:
                    own_partial = partial
                else:
                    psend[l, t - 1, :, :] = partial.astype(jnp.bfloat16)
                    r = pltpu.make_async_remote_copy(
                        src_ref=psend.at[l, t - 1],
                        dst_ref=rsbuf.at[l, t - 1],
                        send_sem=rs_send.at[l, t - 1],
                        recv_sem=rs_recv.at[l, t - 1],
                        device_id=(b,),
                        device_id_type=pl.DeviceIdType.MESH,
                    )
                    r.start()
                    send_handles.append(r)

            if l == 0:
                start_wload(2)

            for s in range(N_DEV - 1):
                pltpu.make_async_remote_copy(
                    src_ref=psend.at[l, s],
                    dst_ref=rsbuf.at[l, s],
                    send_sem=rs_send.at[l, s],
                    recv_sem=rs_recv.at[l, s],
                    device_id=(my,),
                    device_id_type=pl.DeviceIdType.MESH,
                ).wait_recv()
                own_partial = own_partial + rsbuf[l, s, :, :].astype(jnp.float32)

            if l < N_LAYERS - 1:
                xblk0 = own_partial.astype(jnp.bfloat16)
                xbuf[l + 1, my_block, :] = xblk0
                ag_push(l + 1, xbuf.at[l + 1, my_block, :])
            else:
                out_ref[:, :] = own_partial

        for r in send_handles:
            r.wait_send()

    return pl.pallas_call(
        body,
        out_shape=jax.ShapeDtypeStruct((m, d), jnp.float32),
        in_specs=[pl.BlockSpec(memory_space=pltpu.VMEM)]
        + [pl.BlockSpec(memory_space=pl.ANY)] * 6,
        out_specs=pl.BlockSpec(memory_space=pltpu.VMEM),
        scratch_shapes=[
            pltpu.VMEM((N_LAYERS, M, d), jnp.bfloat16),
            pltpu.VMEM((N_LAYERS, N_DEV - 1, m, d), jnp.bfloat16),
            pltpu.VMEM((N_LAYERS, N_DEV - 1, m, d), jnp.bfloat16),
            pltpu.VMEM((2, d, h), jnp.float32),
            pltpu.VMEM((2, h, d), jnp.float32),
            pltpu.SemaphoreType.DMA((N_LAYERS, 2)),
            pltpu.SemaphoreType.DMA((N_LAYERS, N_DEV - 1)),
            pltpu.SemaphoreType.DMA((N_LAYERS, N_DEV - 1)),
            pltpu.SemaphoreType.DMA((N_LAYERS, N_DEV - 1)),
            pltpu.SemaphoreType.DMA((N_LAYERS, N_DEV - 1)),
        ],
        compiler_params=pltpu.CompilerParams(
            collective_id=0,
            vmem_limit_bytes=60 * 1024 * 1024,
        ),
    )(x, Win0, Wout0, Win1, Wout1, Win2, Wout2)
